# Initial kernel scaffold; baseline (speedup 1.0000x reference)
import jax
import jax.numpy as jnp
from jax.experimental import pallas as pl


def kernel(team_obs_rep, tactic_onehot_rep, unit_ids_onehot, W1, b1, W2, b2, pW1, pb1, pW2, pb2, vW1, vb1, vW2, vb2):
    raise NotImplementedError("write your pallas kernel here")


# speedup vs baseline: 1.5104x; 1.5104x over previous
import functools

import jax
import jax.numpy as jnp
from jax import lax
from jax.experimental import pallas as pl
from jax.experimental.pallas import tpu as pltpu

NUM_UNITS = 16
ACTION_DIM = 32


def _dot(a, b):
    return lax.dot_general(a, b, (((1,), (0,)), ((), ())),
                           preferred_element_type=jnp.float32)


def _fused_body(obs_ref, tac_ref, uoh_ref, W1a_ref, W1b_ref, b1_ref, W2_ref,
                b2_ref, pW1_ref, pb1_ref, pW2_ref, pb2_ref, vW1_ref, vb1_ref,
                vW2_ref, vb2_ref, log_ref, val_ref, *, block_rows):
    B = block_rows
    h1 = _dot(obs_ref[...], W1a_ref[...]) + _dot(tac_ref[...], W1b_ref[...])
    h1 = jnp.maximum(h1 + b1_ref[...], 0.0)
    h = jnp.maximum(_dot(h1.astype(jnp.bfloat16), W2_ref[...]) + b2_ref[...], 0.0)
    hb = h.astype(jnp.bfloat16)

    uoh = uoh_ref[...]
    mx = jnp.max(uoh, axis=1, keepdims=True)
    lanes = lax.broadcasted_iota(jnp.int32, (B, NUM_UNITS), 1)
    idx = jnp.min(jnp.where(uoh == mx, lanes, NUM_UNITS), axis=1, keepdims=True)

    lacc = jnp.zeros((B, ACTION_DIM), jnp.float32)
    vacc = jnp.zeros((B, 1), jnp.float32)
    for u in range(NUM_UNITS):
        m = (idx == u).astype(jnp.float32)
        hk = jnp.maximum(_dot(hb, pW1_ref[u]) + pb1_ref[u:u + 1], 0.0)
        lg = _dot(hk.astype(jnp.bfloat16), pW2_ref[u]) + pb2_ref[u:u + 1]
        vk = jnp.maximum(_dot(hb, vW1_ref[u]) + vb1_ref[u:u + 1], 0.0)
        vv = jnp.sum(vk * vW2_ref[u:u + 1, :], axis=1, keepdims=True) + vb2_ref[0, u]
        lacc = lacc + m * lg
        vacc = vacc + m * vv
    log_ref[...] = lacc
    val_ref[...] = vacc


def _fused_call(obs, tac, uoh, W1a, W1b, b1, W2, b2, pW1, pb1, pW2, pb2,
                vW1, vb1, vW2, vb2, *, block_rows, interpret=False):
    N = obs.shape[0]
    grid = (N // block_rows,)
    B = block_rows

    def rows(i):
        return (i, 0)

    def full2(i):
        return (0, 0)

    def full3(i):
        return (0, 0, 0)

    in_specs = [
        pl.BlockSpec((B, 128), rows),
        pl.BlockSpec((B, 16), rows),
        pl.BlockSpec((B, 16), rows),
        pl.BlockSpec(W1a.shape, full2),
        pl.BlockSpec(W1b.shape, full2),
        pl.BlockSpec(b1.shape, full2),
        pl.BlockSpec(W2.shape, full2),
        pl.BlockSpec(b2.shape, full2),
        pl.BlockSpec(pW1.shape, full3),
        pl.BlockSpec(pb1.shape, full2),
        pl.BlockSpec(pW2.shape, full3),
        pl.BlockSpec(pb2.shape, full2),
        pl.BlockSpec(vW1.shape, full3),
        pl.BlockSpec(vb1.shape, full2),
        pl.BlockSpec(vW2.shape, full2),
        pl.BlockSpec(vb2.shape, full2),
    ]
    out_specs = [
        pl.BlockSpec((B, ACTION_DIM), rows),
        pl.BlockSpec((B, 1), rows),
    ]
    out_shape = [
        jax.ShapeDtypeStruct((N, ACTION_DIM), jnp.float32),
        jax.ShapeDtypeStruct((N, 1), jnp.float32),
    ]
    return pl.pallas_call(
        functools.partial(_fused_body, block_rows=B),
        grid=grid,
        in_specs=in_specs,
        out_specs=out_specs,
        out_shape=out_shape,
        interpret=interpret,
    )(obs, tac, uoh, W1a, W1b, b1, W2, b2, pW1, pb1, pW2, pb2, vW1, vb1,
      vW2, vb2)


def kernel(team_obs_rep, tactic_onehot_rep, unit_ids_onehot, W1, b1, W2, b2,
           pW1, pb1, pW2, pb2, vW1, vb1, vW2, vb2, *, block_rows=512,
           interpret=False):
    bf = jnp.bfloat16
    obs = team_obs_rep.astype(bf)
    tac = tactic_onehot_rep.astype(bf)
    W1a = W1[:128].astype(bf)
    W1b = W1[128:].astype(bf)
    logits, values = _fused_call(
        obs, tac, unit_ids_onehot,
        W1a, W1b, b1.reshape(1, -1), W2.astype(bf), b2.reshape(1, -1),
        pW1.astype(bf), pb1, pW2.astype(bf), pb2,
        vW1.astype(bf), vb1, vW2.reshape(NUM_UNITS, -1), vb2.reshape(1, -1),
        block_rows=block_rows, interpret=interpret)
    return logits, values.reshape(-1)
